# Initial kernel scaffold; baseline (speedup 1.0000x reference)
import functools

import jax
import jax.numpy as jnp
from jax import lax
from jax.experimental import pallas as pl
from jax.experimental.pallas import tpu as pltpu
from jax.experimental.pallas import tpu_sc as plsc

B, H, S, D = 2, 32, 4096, 128
BH = B * H
K_TOP = S // 2
HID = D // 2


def _scores_body(k_ref, v_ref, w1_ref, b1_ref, w2_ref, b2_ref, out_ref):
    k = k_ref[0]
    v = v_ref[0]
    key_mag = jnp.sqrt(jnp.sum(k * k, axis=1, keepdims=True))
    val_mag = jnp.sqrt(jnp.sum(v * v, axis=1, keepdims=True))
    mag = (key_mag + val_mag) / 2.0
    c = k + v
    h = lax.dot_general(c, w1_ref[...], (((1,), (1,)), ((), ())))
    h = h + b1_ref[...]
    h = jnp.maximum(h, 0.0)
    o = lax.dot_general(h, w2_ref[...], (((1,), (1,)), ((), ())))
    o = o + b2_ref[...]
    learned = jax.nn.sigmoid(o)
    den = jnp.maximum(jnp.sqrt(jnp.sum(mag * mag)), 1e-12)
    ones_term = jnp.float32(0.2) * (jnp.float32(1.0) / jnp.float32(64.0))
    scores = 0.4 * (mag / den) + 0.4 * learned + ones_term
    out_ref[0] = scores


def _compute_scores(k3, v3, W1, b1, W2, b2):
    b1r = b1.reshape(1, HID)
    b2r = b2.reshape(1, 1)
    out = pl.pallas_call(
        _scores_body,
        grid=(BH,),
        in_specs=[
            pl.BlockSpec((1, S, D), lambda i: (i, 0, 0)),
            pl.BlockSpec((1, S, D), lambda i: (i, 0, 0)),
            pl.BlockSpec((HID, D), lambda i: (0, 0)),
            pl.BlockSpec((1, HID), lambda i: (0, 0)),
            pl.BlockSpec((1, HID), lambda i: (0, 0)),
            pl.BlockSpec((1, 1), lambda i: (0, 0)),
        ],
        out_specs=pl.BlockSpec((1, S, 1), lambda i: (i, 0, 0)),
        out_shape=jax.ShapeDtypeStruct((BH, S, 1), jnp.float32),
    )(k3, v3, W1, b1r, W2, b2r)
    return out.reshape(BH, S)


def _shl(x, j):
    return jnp.concatenate([x[:, j:], x[:, :j]], axis=1)


def _sort_body(s_ref, gidx_ref, mask_ref):
    vals = s_ref[...]
    pos = lax.broadcasted_iota(jnp.int32, (BH, S), 1)
    idx = pos
    for kk in [2 << t for t in range(12)]:
        jj = kk >> 1
        while jj > 0:
            is_lower = (pos & jj) == 0
            pv = jnp.where(is_lower, _shl(vals, jj), _shl(vals, S - jj))
            pi = jnp.where(is_lower, _shl(idx, jj), _shl(idx, S - jj))
            first = (vals > pv) | ((vals == pv) & (idx < pi))
            blk_desc = (pos & kk) == 0
            keep = first ^ (~is_lower) ^ (~blk_desc)
            vals = jnp.where(keep, vals, pv)
            idx = jnp.where(keep, idx, pi)
            jj >>= 1
    t_val = vals[:, K_TOP - 1 : K_TOP]
    t_idx = idx[:, K_TOP - 1 : K_TOP]
    s_orig = s_ref[...]
    mask_ref[...] = jnp.where(
        (s_orig > t_val) | ((s_orig == t_val) & (pos <= t_idx)), 1.0, 0.0
    ).astype(jnp.float32)
    row = lax.broadcasted_iota(jnp.int32, (BH, K_TOP), 0)
    gidx_ref[...] = idx[:, :K_TOP] + row * S


def _sort_and_mask(scores):
    return pl.pallas_call(
        _sort_body,
        out_shape=(
            jax.ShapeDtypeStruct((BH, K_TOP), jnp.int32),
            jax.ShapeDtypeStruct((BH, S), jnp.float32),
        ),
    )(scores)


def _make_gather():
    info = plsc.get_sparse_core_info()
    nw = info.num_cores * info.num_subcores
    total_rows = BH * K_TOP
    rows_per_w = total_rows // nw
    chunk = 128
    n_chunks = rows_per_w // chunk
    mesh = plsc.VectorSubcoreMesh(core_axis_name="c", subcore_axis_name="s")

    @functools.partial(
        pl.kernel,
        mesh=mesh,
        out_type=(
            jax.ShapeDtypeStruct((total_rows, D), jnp.float32),
            jax.ShapeDtypeStruct((total_rows, D), jnp.float32),
        ),
        scratch_types=[
            pltpu.VMEM((n_chunks, chunk), jnp.int32),
            pltpu.VMEM((chunk, D), jnp.float32),
            pltpu.VMEM((chunk, D), jnp.float32),
            pltpu.SemaphoreType.DMA,
            pltpu.SemaphoreType.DMA,
        ],
    )
    def gather(k_hbm, v_hbm, gidx_hbm, out_k, out_v, idx_v, kbuf, vbuf, ksem, vsem):
        cid = lax.axis_index("c")
        sid = lax.axis_index("s")
        wid = sid * info.num_cores + cid
        base = wid * rows_per_w
        pltpu.sync_copy(gidx_hbm.at[pl.ds(wid * n_chunks, n_chunks)], idx_v)

        def body(j, carry):
            ck = pltpu.async_copy(k_hbm.at[idx_v.at[j]], kbuf, ksem)
            cv = pltpu.async_copy(v_hbm.at[idx_v.at[j]], vbuf, vsem)
            ck.wait()
            pltpu.sync_copy(kbuf, out_k.at[pl.ds(base + j * chunk, chunk)])
            cv.wait()
            pltpu.sync_copy(vbuf, out_v.at[pl.ds(base + j * chunk, chunk)])
            return carry

        lax.fori_loop(0, n_chunks, body, 0)

    return gather


_gather_fn = None


def kernel(key_states, value_states, W1, b1, W2, b2):
    global _gather_fn
    if _gather_fn is None:
        _gather_fn = _make_gather()
    k3 = key_states.reshape(BH, S, D)
    v3 = value_states.reshape(BH, S, D)
    scores = _compute_scores(k3, v3, W1, b1, W2, b2)
    gidx, mask = _sort_and_mask(scores)
    k2 = key_states.reshape(BH * S, D)
    v2 = value_states.reshape(BH * S, D)
    gidx2 = gidx.reshape(BH * K_TOP // 128, 128)
    ck, cv = _gather_fn(k2, v2, gidx2)
    compressed_keys = ck.reshape(B, H, K_TOP, D)
    compressed_values = cv.reshape(B, H, K_TOP, D)
    return compressed_keys, compressed_values, mask.reshape(B, H, S)


# speedup vs baseline: 1.2379x; 1.2379x over previous
import functools

import jax
import jax.numpy as jnp
from jax import lax
from jax.experimental import pallas as pl
from jax.experimental.pallas import tpu as pltpu
from jax.experimental.pallas import tpu_sc as plsc

B, H, S, D = 2, 32, 4096, 128
BH = B * H
K_TOP = S // 2
HID = D // 2
SBLK = 512


def _rot8(p, r):
    return jnp.concatenate([p[:, r:8], p[:, 0:r]], axis=1) if r else p


def _row_sumsq(x2, pos):
    p = x2[:, 0:8]
    for j in range(1, 16):
        p = p + x2[:, 8 * j : 8 * j + 8]
    q = p + _rot8(p, 4)
    q = q + _rot8(q, 2)
    q = q + _rot8(q, 1)
    g = (pos // 128) % 8
    out = q[:, 0:1]
    for u in range(1, 8):
        out = jnp.where(g == u, q[:, u : u + 1], out)
    return out


def _scores_body(k_ref, v_ref, w1_ref, b1_ref, w2_ref, b2_ref, mag_ref, lrn_ref):
    k = k_ref[0]
    v = v_ref[0]
    pos = lax.broadcasted_iota(jnp.int32, (SBLK, 1), 0) + pl.program_id(1) * SBLK
    key_mag = jnp.sqrt(_row_sumsq(k * k, pos))
    val_mag = jnp.sqrt(_row_sumsq(v * v, pos))
    mag_ref[0] = (key_mag + val_mag) / 2.0
    c = k + v
    h = lax.dot_general(c, w1_ref[...], (((1,), (1,)), ((), ())))
    h = h + b1_ref[...]
    h = jnp.maximum(h, 0.0)
    o = lax.dot_general(h, w2_ref[...], (((1,), (1,)), ((), ())))[:, 0:1]
    o = o + b2_ref[0]
    lrn_ref[0] = jax.nn.sigmoid(o)


def _compute_parts(k3, v3, W1, b1, W2, b2):
    b1r = b1.reshape(1, HID)
    w2p = jnp.concatenate([W2, jnp.zeros((7, HID), jnp.float32)], axis=0)
    mag, lrn = pl.pallas_call(
        _scores_body,
        grid=(BH, S // SBLK),
        in_specs=[
            pl.BlockSpec((1, SBLK, D), lambda i, j: (i, j, 0)),
            pl.BlockSpec((1, SBLK, D), lambda i, j: (i, j, 0)),
            pl.BlockSpec((HID, D), lambda i, j: (0, 0)),
            pl.BlockSpec((1, HID), lambda i, j: (0, 0)),
            pl.BlockSpec((8, HID), lambda i, j: (0, 0)),
            pl.BlockSpec(memory_space=pltpu.SMEM),
        ],
        out_specs=[
            pl.BlockSpec((1, SBLK, 1), lambda i, j: (i, j, 0)),
            pl.BlockSpec((1, SBLK, 1), lambda i, j: (i, j, 0)),
        ],
        out_shape=[
            jax.ShapeDtypeStruct((BH, S, 1), jnp.float32),
            jax.ShapeDtypeStruct((BH, S, 1), jnp.float32),
        ],
    )(k3, v3, W1, b1r, w2p, b2)
    return mag.reshape(BH, S), lrn.reshape(BH, S)


def _shl(x, j):
    return jnp.concatenate([x[:, j:], x[:, :j]], axis=1)


def _sort_body(mag_ref, lrn_ref, gidx_ref, mask_ref):
    mag = mag_ref[...]
    den = jnp.maximum(jnp.sqrt(jnp.sum(mag * mag, axis=1, keepdims=True)), 1e-12)
    ones_term = jnp.float32(0.2) * jnp.float32(0.015625)
    scores = 0.4 * (mag / den) + 0.4 * lrn_ref[...] + ones_term

    vals = scores
    pos = lax.broadcasted_iota(jnp.int32, (BH, S), 1)
    idx = pos
    for kk in [2 << t for t in range(12)]:
        jj = kk >> 1
        while jj > 0:
            is_lower = (pos & jj) == 0
            pv = jnp.where(is_lower, _shl(vals, jj), _shl(vals, S - jj))
            pi = jnp.where(is_lower, _shl(idx, jj), _shl(idx, S - jj))
            first = (vals > pv) | ((vals == pv) & (idx < pi))
            blk_desc = (pos & kk) == 0
            keep = first ^ (~is_lower) ^ (~blk_desc)
            vals = jnp.where(keep, vals, pv)
            idx = jnp.where(keep, idx, pi)
            jj >>= 1
    t_val = vals[:, K_TOP - 1 : K_TOP]
    t_idx = idx[:, K_TOP - 1 : K_TOP]
    mask_ref[...] = jnp.where(
        (scores > t_val) | ((scores == t_val) & (pos <= t_idx)), 1.0, 0.0
    ).astype(jnp.float32)
    row = lax.broadcasted_iota(jnp.int32, (BH, K_TOP), 0)
    gidx_ref[...] = idx[:, :K_TOP] + row * S


def _sort_and_mask(mag, lrn):
    return pl.pallas_call(
        _sort_body,
        out_shape=(
            jax.ShapeDtypeStruct((BH, K_TOP), jnp.int32),
            jax.ShapeDtypeStruct((BH, S), jnp.float32),
        ),
    )(mag, lrn)


def _make_gather():
    info = plsc.get_sparse_core_info()
    nw = info.num_cores * info.num_subcores
    total_rows = BH * K_TOP
    rows_per_w = total_rows // nw
    chunk = 128
    n_chunks = rows_per_w // chunk
    mesh = plsc.VectorSubcoreMesh(core_axis_name="c", subcore_axis_name="s")

    @functools.partial(
        pl.kernel,
        mesh=mesh,
        out_type=(
            jax.ShapeDtypeStruct((total_rows, D), jnp.float32),
            jax.ShapeDtypeStruct((total_rows, D), jnp.float32),
        ),
        scratch_types=[
            pltpu.VMEM((n_chunks, chunk), jnp.int32),
            pltpu.VMEM((chunk, D), jnp.float32),
            pltpu.VMEM((chunk, D), jnp.float32),
            pltpu.SemaphoreType.DMA,
            pltpu.SemaphoreType.DMA,
        ],
    )
    def gather(k_hbm, v_hbm, gidx_hbm, out_k, out_v, idx_v, kbuf, vbuf, ksem, vsem):
        cid = lax.axis_index("c")
        sid = lax.axis_index("s")
        wid = sid * info.num_cores + cid
        base = wid * rows_per_w
        pltpu.sync_copy(gidx_hbm.at[pl.ds(wid * n_chunks, n_chunks)], idx_v)

        def body(j, carry):
            ck = pltpu.async_copy(k_hbm.at[idx_v.at[j]], kbuf, ksem)
            cv = pltpu.async_copy(v_hbm.at[idx_v.at[j]], vbuf, vsem)
            ck.wait()
            pltpu.sync_copy(kbuf, out_k.at[pl.ds(base + j * chunk, chunk)])
            cv.wait()
            pltpu.sync_copy(vbuf, out_v.at[pl.ds(base + j * chunk, chunk)])
            return carry

        lax.fori_loop(0, n_chunks, body, 0)

    return gather


_gather_fn = None


def kernel(key_states, value_states, W1, b1, W2, b2):
    global _gather_fn
    if _gather_fn is None:
        _gather_fn = _make_gather()
    k3 = key_states.reshape(BH, S, D)
    v3 = value_states.reshape(BH, S, D)
    mag, lrn = _compute_parts(k3, v3, W1, b1, W2, b2)
    gidx, mask = _sort_and_mask(mag, lrn)
    k2 = key_states.reshape(BH * S, D)
    v2 = value_states.reshape(BH * S, D)
    gidx2 = gidx.reshape(BH * K_TOP // 128, 128)
    ck, cv = _gather_fn(k2, v2, gidx2)
    compressed_keys = ck.reshape(B, H, K_TOP, D)
    compressed_values = cv.reshape(B, H, K_TOP, D)
    return compressed_keys, compressed_values, mask.reshape(B, H, S)
